# baseline (speedup 1.0000x reference)
import jax
import jax.numpy as jnp
from jax import lax
from jax.experimental import pallas as pl
from jax.experimental.pallas import tpu as pltpu
from jax.experimental.pallas import tpu_sc as plsc

N = 10000
E = 320000
P = 1024
DF = 128
DG = 128
DE = 16
A = 8

NC = 2
NS = 16
NW = NC * NS
NE_TILE = E // NW
CH = 80
NCH = NE_TILE // CH
NP_ = 10240
ROWS_TILE = NP_ // NS
ZROWS = 128

LCAP = 4096
LSAFE = LCAP - CH
LTOT = NW * LCAP + NW * 128
NP2 = 1152
R2_TILE = NP2 // NS
DEAD = P
SB = 11

_f32 = jnp.float32
_i32 = jnp.int32


def _relu_add_rows(rows_v, ee_v, n_rows):
    def _row(r, _):
        for j in range(DG // 16):
            c = j * 16
            v = rows_v[r, pl.ds(c, 16)] + ee_v[r, pl.ds(c, 16)]
            rows_v[r, pl.ds(c, 16)] = jnp.maximum(v, 0.0)
        return 0
    lax.fori_loop(0, n_rows, _row, 0)


def _zero_rows(zbuf, n_rows):
    def _zr(r, _):
        for j in range(DG // 16):
            zbuf[r, pl.ds(j * 16, 16)] = jnp.zeros((16,), _f32)
        return 0
    lax.fori_loop(0, n_rows, _zr, 0)


_GATHER16_DNUMS = lax.GatherDimensionNumbers(
    offset_dims=(), collapsed_slice_dims=(0,), start_index_map=(0,))


def _gather16(x, idx):
    return lax.gather(x, idx[:, None], _GATHER16_DNUMS, slice_sizes=(1,),
                      mode=lax.GatherScatterMode.PROMISE_IN_BOUNDS)


def _prefix_sum16(x, lane):
    for s in (1, 2, 4, 8):
        sh = _gather16(x, jnp.maximum(lane - s, 0))
        x = x + jnp.where(lane >= s, sh, 0)
    return x


def _sc_step1_body(nf_hbm, s_hbm, r_hbm, ee_hbm,
                   agg_out, eid_out, pkd_out, cnt_out,
                   idx_v, ridx_v, rows_v, ee_v, zbuf,
                   pos_buf, eidb_v, pkdb_v, cntb_v, agg_sh, sem):
    cid = lax.axis_index("c")
    sid = lax.axis_index("s")
    wid = cid * NS + sid

    _zero_rows(zbuf, ZROWS)
    for j in range(ROWS_TILE // ZROWS):
        pltpu.sync_copy(zbuf, agg_sh.at[pl.ds(sid * ROWS_TILE + j * ZROWS, ZROWS)])
    plsc.subcore_barrier()

    lane = lax.iota(_i32, 16)

    def _chunk(k, cnt):
        base = wid * NE_TILE + k * CH
        pltpu.sync_copy(s_hbm.at[pl.ds(base, CH)], idx_v)
        gcp = pltpu.async_copy(nf_hbm.at[idx_v], rows_v, sem)
        pltpu.sync_copy(ee_hbm.at[pl.ds(base, CH)], ee_v)
        pltpu.sync_copy(r_hbm.at[pl.ds(base, CH)], ridx_v)

        for j in range(CH // 16):
            r16 = ridx_v[pl.ds(j * 16, 16)]
            s16 = idx_v[pl.ds(j * 16, 16)]
            e16 = lane + (base + j * 16)
            m = lax.shift_right_logical(r16 - P, 31)
            incl = _prefix_sum16(m, lane)
            posq = wid * LCAP + jnp.minimum(cnt + incl - 1, LCAP - 1)
            post = NW * LCAP + wid * 128 + (lane + j * 16)
            pos_buf[pl.ds(j * 16, 16)] = jnp.where(m == 1, posq, post)
            eidb_v[pl.ds(j * 16, 16)] = e16
            pkdb_v[pl.ds(j * 16, 16)] = lax.shift_left(s16, SB) + r16
            cnt = cnt + incl[15]
        pltpu.sync_copy(eidb_v, eid_out.at[pos_buf])
        pltpu.sync_copy(pkdb_v, pkd_out.at[pos_buf])

        gcp.wait()
        _relu_add_rows(rows_v, ee_v, CH)
        pltpu.sync_copy(rows_v, agg_sh.at[ridx_v], add=True)
        return cnt
    cnt = lax.fori_loop(0, NCH, _chunk, jnp.int32(0))

    zero16 = jnp.zeros((16,), _i32)
    dead16 = jnp.full((16,), DEAD, _i32)
    for j in range(CH // 16):
        pos_buf[pl.ds(j * 16, 16)] = (
            wid * LCAP + jnp.minimum(cnt + j * 16 + lane, LCAP - 1))
        eidb_v[pl.ds(j * 16, 16)] = zero16
        pkdb_v[pl.ds(j * 16, 16)] = dead16
    pltpu.sync_copy(eidb_v, eid_out.at[pos_buf])
    pltpu.sync_copy(pkdb_v, pkd_out.at[pos_buf])
    cntb_v[...] = jnp.full((16,), cnt, _i32)
    pltpu.sync_copy(cntb_v, cnt_out.at[pl.ds(wid * 16, 16)])

    plsc.subcore_barrier()
    pltpu.sync_copy(agg_sh.at[pl.ds(sid * ROWS_TILE, ROWS_TILE)],
                    agg_out.at[cid, pl.ds(sid * ROWS_TILE, ROWS_TILE)])


_sc_step1 = pl.kernel(
    _sc_step1_body,
    out_type=(
        jax.ShapeDtypeStruct((NC, NP_, DG), _f32),
        jax.ShapeDtypeStruct((LTOT,), _i32),
        jax.ShapeDtypeStruct((LTOT,), _i32),
        jax.ShapeDtypeStruct((NW * 16,), _i32),
    ),
    mesh=plsc.VectorSubcoreMesh(core_axis_name="c", subcore_axis_name="s",
                                num_cores=NC, num_subcores=NS),
    scratch_types=[
        pltpu.VMEM((CH,), _i32),
        pltpu.VMEM((CH,), _i32),
        pltpu.VMEM((CH, DG), _f32),
        pltpu.VMEM((CH, DG), _f32),
        pltpu.VMEM((ZROWS, DG), _f32),
        pltpu.VMEM((CH,), _i32),
        pltpu.VMEM((CH,), _i32),
        pltpu.VMEM((CH,), _i32),
        pltpu.VMEM((16,), _i32),
        pltpu.VMEM_SHARED((NP_, DG), _f32),
        pltpu.SemaphoreType.DMA,
    ],
)


def _sc_step2_body(nf_hbm, eid_hbm, pkd_hbm, cnt_hbm, ee_hbm, s_hbm, r_hbm,
                   agg_out,
                   cnt_v, eidx_v, pkd_v, sidx_v, ridx_v, rows_v, ee_v, zbuf,
                   agg_sh, sem, sem2):
    cid = lax.axis_index("c")
    sid = lax.axis_index("s")
    wid = cid * NS + sid

    _zero_rows(zbuf, R2_TILE)
    pltpu.sync_copy(zbuf, agg_sh.at[pl.ds(sid * R2_TILE, R2_TILE)])
    plsc.subcore_barrier()

    pltpu.sync_copy(cnt_hbm.at[pl.ds(wid * 16, 16)], cnt_v)
    c = cnt_v[...][0]
    sat = c > LSAFE
    nch = jnp.where(sat, 0, (c + CH - 1) // CH)
    nch_raw = jnp.where(sat, NCH, 0)

    def _chunk(k, _):
        base = wid * LCAP + k * CH
        pltpu.sync_copy(pkd_hbm.at[pl.ds(base, CH)], pkd_v)
        for j in range(CH // 16):
            pk = pkd_v[pl.ds(j * 16, 16)]
            sidx_v[pl.ds(j * 16, 16)] = lax.shift_right_logical(pk, SB)
            ridx_v[pl.ds(j * 16, 16)] = lax.bitwise_and(pk, (1 << SB) - 1)
        gcp = pltpu.async_copy(nf_hbm.at[sidx_v], rows_v, sem)
        pltpu.sync_copy(eid_hbm.at[pl.ds(base, CH)], eidx_v)
        ecp = pltpu.async_copy(ee_hbm.at[eidx_v], ee_v, sem2)
        gcp.wait()
        ecp.wait()
        _relu_add_rows(rows_v, ee_v, CH)
        pltpu.sync_copy(rows_v, agg_sh.at[ridx_v], add=True)
        return 0
    lax.fori_loop(0, nch, _chunk, 0)

    def _chunk_raw(k, _):
        base = wid * NE_TILE + k * CH
        pltpu.sync_copy(s_hbm.at[pl.ds(base, CH)], sidx_v)
        gcp = pltpu.async_copy(nf_hbm.at[sidx_v], rows_v, sem)
        pltpu.sync_copy(ee_hbm.at[pl.ds(base, CH)], ee_v)
        pltpu.sync_copy(r_hbm.at[pl.ds(base, CH)], pkd_v)
        for j in range(CH // 16):
            r16 = pkd_v[pl.ds(j * 16, 16)]
            ridx_v[pl.ds(j * 16, 16)] = jnp.where(r16 < P, r16, DEAD)
        gcp.wait()
        _relu_add_rows(rows_v, ee_v, CH)
        pltpu.sync_copy(rows_v, agg_sh.at[ridx_v], add=True)
        return 0
    lax.fori_loop(0, nch_raw, _chunk_raw, 0)
    plsc.subcore_barrier()

    pltpu.sync_copy(agg_sh.at[pl.ds(sid * R2_TILE, R2_TILE)],
                    agg_out.at[cid, pl.ds(sid * R2_TILE, R2_TILE)])


_sc_step2 = pl.kernel(
    _sc_step2_body,
    out_type=jax.ShapeDtypeStruct((NC, NP2, DG), _f32),
    mesh=plsc.VectorSubcoreMesh(core_axis_name="c", subcore_axis_name="s",
                                num_cores=NC, num_subcores=NS),
    scratch_types=[
        pltpu.VMEM((16,), _i32),
        pltpu.VMEM((CH,), _i32),
        pltpu.VMEM((CH,), _i32),
        pltpu.VMEM((CH,), _i32),
        pltpu.VMEM((CH,), _i32),
        pltpu.VMEM((CH, DG), _f32),
        pltpu.VMEM((CH, DG), _f32),
        pltpu.VMEM((R2_TILE, DG), _f32),
        pltpu.VMEM_SHARED((NP2, DG), _f32),
        pltpu.SemaphoreType.DMA,
        pltpu.SemaphoreType.DMA,
    ],
)


def _nf_body(x_ref, w_ref, b_ref, o_ref):
    o_ref[...] = jnp.dot(x_ref[...], w_ref[...],
                         preferred_element_type=_f32) + b_ref[...]


def _node_encode(x, w, b2):
    return pl.pallas_call(
        _nf_body,
        out_shape=jax.ShapeDtypeStruct((N, DG), _f32),
    )(x, w, b2)


_EB = 8000


def _edge_proj(ef, w, b2):
    return pl.pallas_call(
        _nf_body,
        grid=(E // _EB,),
        in_specs=[
            pl.BlockSpec((_EB, DE), lambda i: (i, 0)),
            pl.BlockSpec((DE, DG), lambda i: (0, 0)),
            pl.BlockSpec((1, DG), lambda i: (0, 0)),
        ],
        out_specs=pl.BlockSpec((_EB, DG), lambda i: (i, 0)),
        out_shape=jax.ShapeDtypeStruct((E, DG), _f32),
    )(ef, w, b2)


def _mid_body(aggs_ref, nf_ref, w_ref, b_ref, o_ref):
    a = aggs_ref[0] + aggs_ref[1]
    h = jnp.maximum(jnp.dot(a, w_ref[...], preferred_element_type=_f32)
                    + b_ref[...], 0.0)
    o_ref[...] = h + nf_ref[...]


def _mid(aggs, nf, w, b2):
    return pl.pallas_call(
        _mid_body,
        grid=(1,),
        in_specs=[
            pl.BlockSpec((NC, N, DG), lambda i: (0, 0, 0)),
            pl.BlockSpec((N, DG), lambda i: (0, 0)),
            pl.BlockSpec((DG, DG), lambda i: (0, 0)),
            pl.BlockSpec((1, DG), lambda i: (0, 0)),
        ],
        out_specs=pl.BlockSpec((N, DG), lambda i: (0, 0)),
        out_shape=jax.ShapeDtypeStruct((N, DG), _f32),
    )(aggs, nf, w, b2)


def _head_body(q_ref, lat_ref, wg_ref, bg_ref, wd_ref, bd_ref, wh_ref, bh_ref,
               o_ref):
    a = q_ref[0] + q_ref[1]
    l2 = jnp.maximum(jnp.dot(a, wg_ref[...], preferred_element_type=_f32)
                     + bg_ref[...], 0.0)
    dcd = jnp.dot(l2, wd_ref[...], preferred_element_type=_f32) + bd_ref[...]
    cat = jnp.concatenate([lat_ref[...], dcd], axis=-1)
    o_ref[...] = jnp.dot(cat, wh_ref[...], preferred_element_type=_f32) \
        + bh_ref[...]


def _heads(aggs, latents, wg, bg2, wd, bd2, wh, bh2):
    return pl.pallas_call(
        _head_body,
        grid=(1,),
        in_specs=[
            pl.BlockSpec((NC, P, DG), lambda i: (0, 0, 0)),
            pl.BlockSpec((P, DF), lambda i: (0, 0)),
            pl.BlockSpec((DG, DG), lambda i: (0, 0)),
            pl.BlockSpec((1, DG), lambda i: (0, 0)),
            pl.BlockSpec((DG, DG), lambda i: (0, 0)),
            pl.BlockSpec((1, DG), lambda i: (0, 0)),
            pl.BlockSpec((DF + DG, 128), lambda i: (0, 0)),
            pl.BlockSpec((1, 128), lambda i: (0, 0)),
        ],
        out_specs=pl.BlockSpec((P, 128), lambda i: (0, 0)),
        out_shape=jax.ShapeDtypeStruct((P, 128), _f32),
    )(aggs, latents, wg, bg2, wd, bd2, wh, bh2)


def kernel(latents, node_features, senders, receivers, edge_features,
           W_t2g, b_t2g, W_edge, b_edge, W_gnn, b_gnn, W_dec, b_dec,
           W_actor, b_actor, W_critic, b_critic):
    bg2 = b_gnn.reshape(1, DG)
    nf = _node_encode(node_features, W_t2g, b_t2g.reshape(1, DG))
    ee = _edge_proj(edge_features, W_edge, b_edge.reshape(1, DG))

    aggs1, eids, pkds, cnts = _sc_step1(nf, senders, receivers, ee)
    nf2 = _mid(aggs1, nf, W_gnn, bg2)
    aggs2 = _sc_step2(nf2, eids, pkds, cnts, ee, senders, receivers)

    wh = jnp.zeros((DF + DG, 128), _f32)
    wh = wh.at[:, :A].set(W_actor).at[:, A:A + 1].set(W_critic)
    bh = jnp.zeros((1, 128), _f32)
    bh = bh.at[0, :A].set(b_actor).at[0, A:A + 1].set(b_critic)

    heads = _heads(aggs2, latents, W_gnn, bg2, W_dec, b_dec.reshape(1, DG),
                   wh, bh)
    value = heads[:, A:A + 1]
    policy = heads[:, :A]
    return value, policy


# speedup vs baseline: 4.4748x; 4.4748x over previous
import jax
import jax.numpy as jnp
from jax import lax
from jax.experimental import pallas as pl
from jax.experimental.pallas import tpu as pltpu
from jax.experimental.pallas import tpu_sc as plsc

N = 10000
E = 320000
P = 1024
DF = 128
DG = 128
DE = 16
A = 8

NC = 2
NS = 16
NW = NC * NS
NE_TILE = E // NW
CH = 80
NCH = NE_TILE // CH
NP_ = 10240
ROWS_TILE = NP_ // NS
ZROWS = 128

LCAP = 4096
LSAFE = LCAP - CH
LTOT = NW * LCAP
LSP = NS * (LCAP + 128)
NP2 = 1152
R2_TILE = NP2 // NS
DEAD = P
SB = 11

_f32 = jnp.float32
_i32 = jnp.int32


def _relu_add_rows(rows_v, ee_v, n_rows):
    def _row(r, _):
        for j in range(DG // 16):
            c = j * 16
            v = rows_v[r, pl.ds(c, 16)] + ee_v[r, pl.ds(c, 16)]
            rows_v[r, pl.ds(c, 16)] = jnp.maximum(v, 0.0)
        return 0
    lax.fori_loop(0, n_rows, _row, 0)


def _zero_rows(zbuf, n_rows):
    def _zr(r, _):
        for j in range(DG // 16):
            zbuf[r, pl.ds(j * 16, 16)] = jnp.zeros((16,), _f32)
        return 0
    lax.fori_loop(0, n_rows, _zr, 0)


_GATHER16_DNUMS = lax.GatherDimensionNumbers(
    offset_dims=(), collapsed_slice_dims=(0,), start_index_map=(0,))


def _gather16(x, idx):
    return lax.gather(x, idx[:, None], _GATHER16_DNUMS, slice_sizes=(1,),
                      mode=lax.GatherScatterMode.PROMISE_IN_BOUNDS)


def _prefix_sum16(x, lane):
    for s in (1, 2, 4, 8):
        sh = _gather16(x, jnp.maximum(lane - s, 0))
        x = x + jnp.where(lane >= s, sh, 0)
    return x


def _sc_step1_body(nf_hbm, s_hbm, r_hbm, ee_hbm,
                   agg_out, eid_out, pkd_out, cnt_out,
                   idx_v, ridx_v, rows_v, ee_v, zbuf,
                   pos_buf, eidb_v, pkdb_v, cntb_v, agg_sh, eid_sp, pkd_sp,
                   sem):
    cid = lax.axis_index("c")
    sid = lax.axis_index("s")
    wid = cid * NS + sid

    _zero_rows(zbuf, ZROWS)
    for j in range(ROWS_TILE // ZROWS):
        pltpu.sync_copy(zbuf, agg_sh.at[pl.ds(sid * ROWS_TILE + j * ZROWS, ZROWS)])
    plsc.subcore_barrier()

    lane = lax.iota(_i32, 16)

    def _chunk(k, cnt):
        base = wid * NE_TILE + k * CH
        pltpu.sync_copy(s_hbm.at[pl.ds(base, CH)], idx_v)
        gcp = pltpu.async_copy(nf_hbm.at[idx_v], rows_v, sem)
        pltpu.sync_copy(ee_hbm.at[pl.ds(base, CH)], ee_v)
        pltpu.sync_copy(r_hbm.at[pl.ds(base, CH)], ridx_v)

        for j in range(CH // 16):
            r16 = ridx_v[pl.ds(j * 16, 16)]
            s16 = idx_v[pl.ds(j * 16, 16)]
            e16 = lane + (base + j * 16)
            m = lax.shift_right_logical(r16 - P, 31)
            incl = _prefix_sum16(m, lane)
            posq = sid * LCAP + jnp.minimum(cnt + incl - 1, LCAP - 1)
            post = NS * LCAP + sid * 128 + (lane + j * 16)
            pos_buf[pl.ds(j * 16, 16)] = jnp.where(m == 1, posq, post)
            eidb_v[pl.ds(j * 16, 16)] = e16
            pkdb_v[pl.ds(j * 16, 16)] = lax.shift_left(s16, SB) + r16
            cnt = cnt + incl[15]
        pltpu.sync_copy(eidb_v, eid_sp.at[pos_buf])
        pltpu.sync_copy(pkdb_v, pkd_sp.at[pos_buf])

        gcp.wait()
        _relu_add_rows(rows_v, ee_v, CH)
        pltpu.sync_copy(rows_v, agg_sh.at[ridx_v], add=True)
        return cnt
    cnt = lax.fori_loop(0, NCH, _chunk, jnp.int32(0))

    zero16 = jnp.zeros((16,), _i32)
    dead16 = jnp.full((16,), DEAD, _i32)
    for j in range(CH // 16):
        pos_buf[pl.ds(j * 16, 16)] = (
            sid * LCAP + jnp.minimum(cnt + j * 16 + lane, LCAP - 1))
        eidb_v[pl.ds(j * 16, 16)] = zero16
        pkdb_v[pl.ds(j * 16, 16)] = dead16
    pltpu.sync_copy(eidb_v, eid_sp.at[pos_buf])
    pltpu.sync_copy(pkdb_v, pkd_sp.at[pos_buf])

    pltpu.sync_copy(eid_sp.at[pl.ds(sid * LCAP, LCAP)],
                    eid_out.at[pl.ds(wid * LCAP, LCAP)])
    pltpu.sync_copy(pkd_sp.at[pl.ds(sid * LCAP, LCAP)],
                    pkd_out.at[pl.ds(wid * LCAP, LCAP)])
    cntb_v[...] = jnp.full((16,), cnt, _i32)
    pltpu.sync_copy(cntb_v, cnt_out.at[pl.ds(wid * 16, 16)])

    plsc.subcore_barrier()
    pltpu.sync_copy(agg_sh.at[pl.ds(sid * ROWS_TILE, ROWS_TILE)],
                    agg_out.at[cid, pl.ds(sid * ROWS_TILE, ROWS_TILE)])


_sc_step1 = pl.kernel(
    _sc_step1_body,
    out_type=(
        jax.ShapeDtypeStruct((NC, NP_, DG), _f32),
        jax.ShapeDtypeStruct((LTOT,), _i32),
        jax.ShapeDtypeStruct((LTOT,), _i32),
        jax.ShapeDtypeStruct((NW * 16,), _i32),
    ),
    mesh=plsc.VectorSubcoreMesh(core_axis_name="c", subcore_axis_name="s",
                                num_cores=NC, num_subcores=NS),
    scratch_types=[
        pltpu.VMEM((CH,), _i32),
        pltpu.VMEM((CH,), _i32),
        pltpu.VMEM((CH, DG), _f32),
        pltpu.VMEM((CH, DG), _f32),
        pltpu.VMEM((ZROWS, DG), _f32),
        pltpu.VMEM((CH,), _i32),
        pltpu.VMEM((CH,), _i32),
        pltpu.VMEM((CH,), _i32),
        pltpu.VMEM((16,), _i32),
        pltpu.VMEM_SHARED((NP_, DG), _f32),
        pltpu.VMEM_SHARED((LSP,), _i32),
        pltpu.VMEM_SHARED((LSP,), _i32),
        pltpu.SemaphoreType.DMA,
    ],
)


def _sc_step2_body(nf_hbm, eid_hbm, pkd_hbm, cnt_hbm, ee_hbm, s_hbm, r_hbm,
                   agg_out,
                   cnt_v, eidx_v, pkd_v, sidx_v, ridx_v, rows_v, ee_v, zbuf,
                   agg_sh, sem, sem2):
    cid = lax.axis_index("c")
    sid = lax.axis_index("s")
    wid = cid * NS + sid

    _zero_rows(zbuf, R2_TILE)
    pltpu.sync_copy(zbuf, agg_sh.at[pl.ds(sid * R2_TILE, R2_TILE)])
    plsc.subcore_barrier()

    pltpu.sync_copy(cnt_hbm.at[pl.ds(wid * 16, 16)], cnt_v)
    c = cnt_v[...][0]
    sat = c > LSAFE
    nch = jnp.where(sat, 0, (c + CH - 1) // CH)
    nch_raw = jnp.where(sat, NCH, 0)

    def _chunk(k, _):
        base = wid * LCAP + k * CH
        pltpu.sync_copy(pkd_hbm.at[pl.ds(base, CH)], pkd_v)
        for j in range(CH // 16):
            pk = pkd_v[pl.ds(j * 16, 16)]
            sidx_v[pl.ds(j * 16, 16)] = lax.shift_right_logical(pk, SB)
            ridx_v[pl.ds(j * 16, 16)] = lax.bitwise_and(pk, (1 << SB) - 1)
        gcp = pltpu.async_copy(nf_hbm.at[sidx_v], rows_v, sem)
        pltpu.sync_copy(eid_hbm.at[pl.ds(base, CH)], eidx_v)
        ecp = pltpu.async_copy(ee_hbm.at[eidx_v], ee_v, sem2)
        gcp.wait()
        ecp.wait()
        _relu_add_rows(rows_v, ee_v, CH)
        pltpu.sync_copy(rows_v, agg_sh.at[ridx_v], add=True)
        return 0
    lax.fori_loop(0, nch, _chunk, 0)

    def _chunk_raw(k, _):
        base = wid * NE_TILE + k * CH
        pltpu.sync_copy(s_hbm.at[pl.ds(base, CH)], sidx_v)
        gcp = pltpu.async_copy(nf_hbm.at[sidx_v], rows_v, sem)
        pltpu.sync_copy(ee_hbm.at[pl.ds(base, CH)], ee_v)
        pltpu.sync_copy(r_hbm.at[pl.ds(base, CH)], pkd_v)
        for j in range(CH // 16):
            r16 = pkd_v[pl.ds(j * 16, 16)]
            ridx_v[pl.ds(j * 16, 16)] = jnp.where(r16 < P, r16, DEAD)
        gcp.wait()
        _relu_add_rows(rows_v, ee_v, CH)
        pltpu.sync_copy(rows_v, agg_sh.at[ridx_v], add=True)
        return 0
    lax.fori_loop(0, nch_raw, _chunk_raw, 0)
    plsc.subcore_barrier()

    pltpu.sync_copy(agg_sh.at[pl.ds(sid * R2_TILE, R2_TILE)],
                    agg_out.at[cid, pl.ds(sid * R2_TILE, R2_TILE)])


_sc_step2 = pl.kernel(
    _sc_step2_body,
    out_type=jax.ShapeDtypeStruct((NC, NP2, DG), _f32),
    mesh=plsc.VectorSubcoreMesh(core_axis_name="c", subcore_axis_name="s",
                                num_cores=NC, num_subcores=NS),
    scratch_types=[
        pltpu.VMEM((16,), _i32),
        pltpu.VMEM((CH,), _i32),
        pltpu.VMEM((CH,), _i32),
        pltpu.VMEM((CH,), _i32),
        pltpu.VMEM((CH,), _i32),
        pltpu.VMEM((CH, DG), _f32),
        pltpu.VMEM((CH, DG), _f32),
        pltpu.VMEM((R2_TILE, DG), _f32),
        pltpu.VMEM_SHARED((NP2, DG), _f32),
        pltpu.SemaphoreType.DMA,
        pltpu.SemaphoreType.DMA,
    ],
)


def _nf_body(x_ref, w_ref, b_ref, o_ref):
    o_ref[...] = jnp.dot(x_ref[...], w_ref[...],
                         preferred_element_type=_f32) + b_ref[...]


def _node_encode(x, w, b2):
    return pl.pallas_call(
        _nf_body,
        out_shape=jax.ShapeDtypeStruct((N, DG), _f32),
    )(x, w, b2)


_EB = 8000


def _edge_proj(ef, w, b2):
    return pl.pallas_call(
        _nf_body,
        grid=(E // _EB,),
        in_specs=[
            pl.BlockSpec((_EB, DE), lambda i: (i, 0)),
            pl.BlockSpec((DE, DG), lambda i: (0, 0)),
            pl.BlockSpec((1, DG), lambda i: (0, 0)),
        ],
        out_specs=pl.BlockSpec((_EB, DG), lambda i: (i, 0)),
        out_shape=jax.ShapeDtypeStruct((E, DG), _f32),
    )(ef, w, b2)


def _mid_body(aggs_ref, nf_ref, w_ref, b_ref, o_ref):
    a = aggs_ref[0] + aggs_ref[1]
    h = jnp.maximum(jnp.dot(a, w_ref[...], preferred_element_type=_f32)
                    + b_ref[...], 0.0)
    o_ref[...] = h + nf_ref[...]


def _mid(aggs, nf, w, b2):
    return pl.pallas_call(
        _mid_body,
        grid=(1,),
        in_specs=[
            pl.BlockSpec((NC, N, DG), lambda i: (0, 0, 0)),
            pl.BlockSpec((N, DG), lambda i: (0, 0)),
            pl.BlockSpec((DG, DG), lambda i: (0, 0)),
            pl.BlockSpec((1, DG), lambda i: (0, 0)),
        ],
        out_specs=pl.BlockSpec((N, DG), lambda i: (0, 0)),
        out_shape=jax.ShapeDtypeStruct((N, DG), _f32),
    )(aggs, nf, w, b2)


def _head_body(q_ref, lat_ref, wg_ref, bg_ref, wd_ref, bd_ref, wh_ref, bh_ref,
               o_ref):
    a = q_ref[0] + q_ref[1]
    l2 = jnp.maximum(jnp.dot(a, wg_ref[...], preferred_element_type=_f32)
                     + bg_ref[...], 0.0)
    dcd = jnp.dot(l2, wd_ref[...], preferred_element_type=_f32) + bd_ref[...]
    cat = jnp.concatenate([lat_ref[...], dcd], axis=-1)
    o_ref[...] = jnp.dot(cat, wh_ref[...], preferred_element_type=_f32) \
        + bh_ref[...]


def _heads(aggs, latents, wg, bg2, wd, bd2, wh, bh2):
    return pl.pallas_call(
        _head_body,
        grid=(1,),
        in_specs=[
            pl.BlockSpec((NC, P, DG), lambda i: (0, 0, 0)),
            pl.BlockSpec((P, DF), lambda i: (0, 0)),
            pl.BlockSpec((DG, DG), lambda i: (0, 0)),
            pl.BlockSpec((1, DG), lambda i: (0, 0)),
            pl.BlockSpec((DG, DG), lambda i: (0, 0)),
            pl.BlockSpec((1, DG), lambda i: (0, 0)),
            pl.BlockSpec((DF + DG, 128), lambda i: (0, 0)),
            pl.BlockSpec((1, 128), lambda i: (0, 0)),
        ],
        out_specs=pl.BlockSpec((P, 128), lambda i: (0, 0)),
        out_shape=jax.ShapeDtypeStruct((P, 128), _f32),
    )(aggs, latents, wg, bg2, wd, bd2, wh, bh2)


def kernel(latents, node_features, senders, receivers, edge_features,
           W_t2g, b_t2g, W_edge, b_edge, W_gnn, b_gnn, W_dec, b_dec,
           W_actor, b_actor, W_critic, b_critic):
    bg2 = b_gnn.reshape(1, DG)
    nf = _node_encode(node_features, W_t2g, b_t2g.reshape(1, DG))
    ee = _edge_proj(edge_features, W_edge, b_edge.reshape(1, DG))

    aggs1, eids, pkds, cnts = _sc_step1(nf, senders, receivers, ee)
    nf2 = _mid(aggs1, nf, W_gnn, bg2)
    aggs2 = _sc_step2(nf2, eids, pkds, cnts, ee, senders, receivers)

    wh = jnp.zeros((DF + DG, 128), _f32)
    wh = wh.at[:, :A].set(W_actor).at[:, A:A + 1].set(W_critic)
    bh = jnp.zeros((1, 128), _f32)
    bh = bh.at[0, :A].set(b_actor).at[0, A:A + 1].set(b_critic)

    heads = _heads(aggs2, latents, W_gnn, bg2, W_dec, b_dec.reshape(1, DG),
                   wh, bh)
    value = heads[:, A:A + 1]
    policy = heads[:, :A]
    return value, policy


# speedup vs baseline: 5.8502x; 1.3074x over previous
import jax
import jax.numpy as jnp
from jax import lax
from jax.experimental import pallas as pl
from jax.experimental.pallas import tpu as pltpu
from jax.experimental.pallas import tpu_sc as plsc

N = 10000
E = 320000
P = 1024
DF = 128
DG = 128
DE = 16
A = 8

NC = 2
NS = 16
NW = NC * NS
NE_TILE = E // NW
CH = 80
NCH = NE_TILE // CH
NP_ = 10240
ROWS_TILE = NP_ // NS
ZROWS = 128

LCAP = 2048
LSAFE = LCAP - CH
LTOT = NW * LCAP
LSP = NS * (LCAP + 128)
NP2 = 1152
R2_TILE = NP2 // NS
DEAD = P
SB = 11

_f32 = jnp.float32
_i32 = jnp.int32


def _relu_add_rows(rows_v, ee_v, n_rows):
    def _row(r, _):
        for j in range(DG // 16):
            c = j * 16
            v = rows_v[r, pl.ds(c, 16)] + ee_v[r, pl.ds(c, 16)]
            rows_v[r, pl.ds(c, 16)] = jnp.maximum(v, 0.0)
        return 0
    lax.fori_loop(0, n_rows, _row, 0)


def _zero_rows(zbuf, n_rows):
    def _zr(r, _):
        for j in range(DG // 16):
            zbuf[r, pl.ds(j * 16, 16)] = jnp.zeros((16,), _f32)
        return 0
    lax.fori_loop(0, n_rows, _zr, 0)


_GATHER16_DNUMS = lax.GatherDimensionNumbers(
    offset_dims=(), collapsed_slice_dims=(0,), start_index_map=(0,))


def _gather16(x, idx):
    return lax.gather(x, idx[:, None], _GATHER16_DNUMS, slice_sizes=(1,),
                      mode=lax.GatherScatterMode.PROMISE_IN_BOUNDS)


def _prefix_sum16(x, lane):
    for s in (1, 2, 4, 8):
        sh = _gather16(x, jnp.maximum(lane - s, 0))
        x = x + jnp.where(lane >= s, sh, 0)
    return x


def _sc_step1_body(nf_hbm, s_hbm, r_hbm, ee_hbm,
                   agg_out, eid_out, pkd_out, cnt_out,
                   idx_a, ridx_a, rows_a, ee_a,
                   idx_b, ridx_b, rows_b, ee_b,
                   sbuf_a, sbuf_b, pos_buf, eidb_v, pkdb_v, cntb_v, agg_sh,
                   eid_sp, pkd_sp,
                   semi_a, semr_a, seme_a, semg_a,
                   semi_b, semr_b, seme_b, semg_b, sems_a, sems_b):
    cid = lax.axis_index("c")
    sid = lax.axis_index("s")
    wid = cid * NS + sid
    tbase = wid * NE_TILE

    _zero_rows(rows_a, CH)
    for j in range(ROWS_TILE // CH):
        pltpu.sync_copy(rows_a, agg_sh.at[pl.ds(sid * ROWS_TILE + j * CH, CH)])
    plsc.subcore_barrier()

    lane = lax.iota(_i32, 16)

    BUF_A = (idx_a, ridx_a, rows_a, ee_a, sbuf_a,
             semi_a, semr_a, seme_a, semg_a, sems_a)
    BUF_B = (idx_b, ridx_b, rows_b, ee_b, sbuf_b,
             semi_b, semr_b, seme_b, semg_b, sems_b)

    def _issue_loads(k, buf):
        idxv, ridxv, _, eev, _, semi, semr, seme, _, _ = buf
        base = tbase + k * CH
        pltpu.async_copy(s_hbm.at[pl.ds(base, CH)], idxv, semi)
        pltpu.async_copy(r_hbm.at[pl.ds(base, CH)], ridxv, semr)
        pltpu.async_copy(ee_hbm.at[pl.ds(base, CH)], eev, seme)

    def _wait(src, dst, sem):
        pltpu.make_async_copy(src, dst, sem).wait()

    def _stage(k, cnt, cur, nxt, issue_next, wait_nxt_scatter=True):
        idxv, ridxv, rowsv, eev, sbuf, semi, semr, seme, semg, sems = cur
        base = tbase + k * CH
        if issue_next:
            _issue_loads(k + 1, nxt)

        _wait(r_hbm.at[pl.ds(base, CH)], ridxv, semr)
        for j in range(CH // 16):
            r16 = ridxv[pl.ds(j * 16, 16)]
            s16 = idxv[pl.ds(j * 16, 16)]
            e16 = lane + (base + j * 16)
            sbuf[pl.ds(j * 16, 16)] = r16
            m = lax.shift_right_logical(r16 - P, 31)
            incl = _prefix_sum16(m, lane)
            posq = sid * LCAP + jnp.minimum(cnt + incl - 1, LCAP - 1)
            post = NS * LCAP + sid * 128 + (lane + j * 16)
            pos_buf[pl.ds(j * 16, 16)] = jnp.where(m == 1, posq, post)
            eidb_v[pl.ds(j * 16, 16)] = e16
            pkdb_v[pl.ds(j * 16, 16)] = lax.shift_left(s16, SB) + r16
            cnt = cnt + incl[15]
        pltpu.sync_copy(eidb_v, eid_sp.at[pos_buf])
        pltpu.sync_copy(pkdb_v, pkd_sp.at[pos_buf])

        _wait(nf_hbm.at[idxv], rowsv, semg)
        _wait(ee_hbm.at[pl.ds(base, CH)], eev, seme)
        _relu_add_rows(rowsv, eev, CH)
        pltpu.async_copy(rowsv, agg_sh.at[sbuf], sems, add=True)

        if issue_next:
            idxn, _, rowsn, _, sbufn, semin, _, _, semgn, semsn = nxt
            _wait(s_hbm.at[pl.ds(base + CH, CH)], idxn, semin)
            if wait_nxt_scatter:
                _wait(rowsn, agg_sh.at[sbufn], semsn)
            pltpu.async_copy(nf_hbm.at[idxn], rowsn, semgn)
        return cnt

    _issue_loads(0, BUF_A)
    _wait(s_hbm.at[pl.ds(tbase, CH)], idx_a, semi_a)
    pltpu.async_copy(nf_hbm.at[idx_a], rows_a, semg_a)

    cnt = _stage(0, jnp.int32(0), BUF_A, BUF_B, True, wait_nxt_scatter=False)

    def _pair(g, cnt):
        k = g * 2 + 1
        cnt = _stage(k, cnt, BUF_B, BUF_A, True)
        cnt = _stage(k + 1, cnt, BUF_A, BUF_B, True)
        return cnt
    cnt = lax.fori_loop(0, (NCH - 3) // 2, _pair, cnt)
    cnt = _stage(NCH - 2, cnt, BUF_B, BUF_A, True)
    cnt = _stage(NCH - 1, cnt, BUF_A, BUF_B, False)

    _wait(rows_b, agg_sh.at[sbuf_b], sems_b)
    _wait(rows_a, agg_sh.at[sbuf_a], sems_a)

    zero16 = jnp.zeros((16,), _i32)
    dead16 = jnp.full((16,), DEAD, _i32)
    for j in range(CH // 16):
        pos_buf[pl.ds(j * 16, 16)] = (
            sid * LCAP + jnp.minimum(cnt + j * 16 + lane, LCAP - 1))
        eidb_v[pl.ds(j * 16, 16)] = zero16
        pkdb_v[pl.ds(j * 16, 16)] = dead16
    pltpu.sync_copy(eidb_v, eid_sp.at[pos_buf])
    pltpu.sync_copy(pkdb_v, pkd_sp.at[pos_buf])

    pltpu.sync_copy(eid_sp.at[pl.ds(sid * LCAP, LCAP)],
                    eid_out.at[pl.ds(wid * LCAP, LCAP)])
    pltpu.sync_copy(pkd_sp.at[pl.ds(sid * LCAP, LCAP)],
                    pkd_out.at[pl.ds(wid * LCAP, LCAP)])
    cntb_v[...] = jnp.full((16,), cnt, _i32)
    pltpu.sync_copy(cntb_v, cnt_out.at[pl.ds(wid * 16, 16)])

    plsc.subcore_barrier()
    pltpu.sync_copy(agg_sh.at[pl.ds(sid * ROWS_TILE, ROWS_TILE)],
                    agg_out.at[cid, pl.ds(sid * ROWS_TILE, ROWS_TILE)])


_sc_step1 = pl.kernel(
    _sc_step1_body,
    out_type=(
        jax.ShapeDtypeStruct((NC, NP_, DG), _f32),
        jax.ShapeDtypeStruct((LTOT,), _i32),
        jax.ShapeDtypeStruct((LTOT,), _i32),
        jax.ShapeDtypeStruct((NW * 16,), _i32),
    ),
    mesh=plsc.VectorSubcoreMesh(core_axis_name="c", subcore_axis_name="s",
                                num_cores=NC, num_subcores=NS),
    scratch_types=[
        pltpu.VMEM((CH,), _i32),
        pltpu.VMEM((CH,), _i32),
        pltpu.VMEM((CH, DG), _f32),
        pltpu.VMEM((CH, DG), _f32),
        pltpu.VMEM((CH,), _i32),
        pltpu.VMEM((CH,), _i32),
        pltpu.VMEM((CH, DG), _f32),
        pltpu.VMEM((CH, DG), _f32),
        pltpu.VMEM((CH,), _i32),
        pltpu.VMEM((CH,), _i32),
        pltpu.VMEM((CH,), _i32),
        pltpu.VMEM((CH,), _i32),
        pltpu.VMEM((CH,), _i32),
        pltpu.VMEM((16,), _i32),
        pltpu.VMEM_SHARED((NP_, DG), _f32),
        pltpu.VMEM_SHARED((LSP,), _i32),
        pltpu.VMEM_SHARED((LSP,), _i32),
        pltpu.SemaphoreType.DMA,
        pltpu.SemaphoreType.DMA,
        pltpu.SemaphoreType.DMA,
        pltpu.SemaphoreType.DMA,
        pltpu.SemaphoreType.DMA,
        pltpu.SemaphoreType.DMA,
        pltpu.SemaphoreType.DMA,
        pltpu.SemaphoreType.DMA,
        pltpu.SemaphoreType.DMA,
        pltpu.SemaphoreType.DMA,
    ],
)


def _sc_step2_body(nf_hbm, eid_hbm, pkd_hbm, cnt_hbm, ee_hbm, s_hbm, r_hbm,
                   agg_out,
                   cnt_v, eidx_v, pkd_v, sidx_v, ridx_v, rows_v, ee_v,
                   agg_sh, sem, sem2):
    cid = lax.axis_index("c")
    sid = lax.axis_index("s")
    wid = cid * NS + sid

    _zero_rows(rows_v, R2_TILE)
    pltpu.sync_copy(rows_v.at[pl.ds(0, R2_TILE)],
                    agg_sh.at[pl.ds(sid * R2_TILE, R2_TILE)])
    plsc.subcore_barrier()

    pltpu.sync_copy(cnt_hbm.at[pl.ds(wid * 16, 16)], cnt_v)
    c = cnt_v[...][0]
    sat = c > LSAFE
    nch = jnp.where(sat, 0, (c + CH - 1) // CH)
    nch_raw = jnp.where(sat, NCH, 0)

    def _chunk(k, _):
        base = wid * LCAP + k * CH
        pltpu.sync_copy(pkd_hbm.at[pl.ds(base, CH)], pkd_v)
        for j in range(CH // 16):
            pk = pkd_v[pl.ds(j * 16, 16)]
            sidx_v[pl.ds(j * 16, 16)] = lax.shift_right_logical(pk, SB)
            ridx_v[pl.ds(j * 16, 16)] = lax.bitwise_and(pk, (1 << SB) - 1)
        gcp = pltpu.async_copy(nf_hbm.at[sidx_v], rows_v, sem)
        pltpu.sync_copy(eid_hbm.at[pl.ds(base, CH)], eidx_v)
        ecp = pltpu.async_copy(ee_hbm.at[eidx_v], ee_v, sem2)
        gcp.wait()
        ecp.wait()
        _relu_add_rows(rows_v, ee_v, CH)
        pltpu.sync_copy(rows_v, agg_sh.at[ridx_v], add=True)
        return 0
    lax.fori_loop(0, nch, _chunk, 0)

    def _chunk_raw(k, _):
        base = wid * NE_TILE + k * CH
        pltpu.sync_copy(s_hbm.at[pl.ds(base, CH)], sidx_v)
        gcp = pltpu.async_copy(nf_hbm.at[sidx_v], rows_v, sem)
        pltpu.sync_copy(ee_hbm.at[pl.ds(base, CH)], ee_v)
        pltpu.sync_copy(r_hbm.at[pl.ds(base, CH)], pkd_v)
        for j in range(CH // 16):
            r16 = pkd_v[pl.ds(j * 16, 16)]
            ridx_v[pl.ds(j * 16, 16)] = jnp.where(r16 < P, r16, DEAD)
        gcp.wait()
        _relu_add_rows(rows_v, ee_v, CH)
        pltpu.sync_copy(rows_v, agg_sh.at[ridx_v], add=True)
        return 0
    lax.fori_loop(0, nch_raw, _chunk_raw, 0)
    plsc.subcore_barrier()

    pltpu.sync_copy(agg_sh.at[pl.ds(sid * R2_TILE, R2_TILE)],
                    agg_out.at[cid, pl.ds(sid * R2_TILE, R2_TILE)])


_sc_step2 = pl.kernel(
    _sc_step2_body,
    out_type=jax.ShapeDtypeStruct((NC, NP2, DG), _f32),
    mesh=plsc.VectorSubcoreMesh(core_axis_name="c", subcore_axis_name="s",
                                num_cores=NC, num_subcores=NS),
    scratch_types=[
        pltpu.VMEM((16,), _i32),
        pltpu.VMEM((CH,), _i32),
        pltpu.VMEM((CH,), _i32),
        pltpu.VMEM((CH,), _i32),
        pltpu.VMEM((CH,), _i32),
        pltpu.VMEM((CH, DG), _f32),
        pltpu.VMEM((CH, DG), _f32),
        pltpu.VMEM_SHARED((NP2, DG), _f32),
        pltpu.SemaphoreType.DMA,
        pltpu.SemaphoreType.DMA,
    ],
)


def _nf_body(x_ref, w_ref, b_ref, o_ref):
    o_ref[...] = jnp.dot(x_ref[...], w_ref[...],
                         preferred_element_type=_f32) + b_ref[...]


def _node_encode(x, w, b2):
    return pl.pallas_call(
        _nf_body,
        out_shape=jax.ShapeDtypeStruct((N, DG), _f32),
    )(x, w, b2)


_EB = 8000


def _edge_proj(ef, w, b2):
    return pl.pallas_call(
        _nf_body,
        grid=(E // _EB,),
        in_specs=[
            pl.BlockSpec((_EB, DE), lambda i: (i, 0)),
            pl.BlockSpec((DE, DG), lambda i: (0, 0)),
            pl.BlockSpec((1, DG), lambda i: (0, 0)),
        ],
        out_specs=pl.BlockSpec((_EB, DG), lambda i: (i, 0)),
        out_shape=jax.ShapeDtypeStruct((E, DG), _f32),
    )(ef, w, b2)


def _mid_body(aggs_ref, nf_ref, w_ref, b_ref, o_ref):
    a = aggs_ref[0] + aggs_ref[1]
    h = jnp.maximum(jnp.dot(a, w_ref[...], preferred_element_type=_f32)
                    + b_ref[...], 0.0)
    o_ref[...] = h + nf_ref[...]


def _mid(aggs, nf, w, b2):
    return pl.pallas_call(
        _mid_body,
        grid=(1,),
        in_specs=[
            pl.BlockSpec((NC, N, DG), lambda i: (0, 0, 0)),
            pl.BlockSpec((N, DG), lambda i: (0, 0)),
            pl.BlockSpec((DG, DG), lambda i: (0, 0)),
            pl.BlockSpec((1, DG), lambda i: (0, 0)),
        ],
        out_specs=pl.BlockSpec((N, DG), lambda i: (0, 0)),
        out_shape=jax.ShapeDtypeStruct((N, DG), _f32),
    )(aggs, nf, w, b2)


def _head_body(q_ref, lat_ref, wg_ref, bg_ref, wd_ref, bd_ref, wh_ref, bh_ref,
               o_ref):
    a = q_ref[0] + q_ref[1]
    l2 = jnp.maximum(jnp.dot(a, wg_ref[...], preferred_element_type=_f32)
                     + bg_ref[...], 0.0)
    dcd = jnp.dot(l2, wd_ref[...], preferred_element_type=_f32) + bd_ref[...]
    cat = jnp.concatenate([lat_ref[...], dcd], axis=-1)
    o_ref[...] = jnp.dot(cat, wh_ref[...], preferred_element_type=_f32) \
        + bh_ref[...]


def _heads(aggs, latents, wg, bg2, wd, bd2, wh, bh2):
    return pl.pallas_call(
        _head_body,
        grid=(1,),
        in_specs=[
            pl.BlockSpec((NC, P, DG), lambda i: (0, 0, 0)),
            pl.BlockSpec((P, DF), lambda i: (0, 0)),
            pl.BlockSpec((DG, DG), lambda i: (0, 0)),
            pl.BlockSpec((1, DG), lambda i: (0, 0)),
            pl.BlockSpec((DG, DG), lambda i: (0, 0)),
            pl.BlockSpec((1, DG), lambda i: (0, 0)),
            pl.BlockSpec((DF + DG, 128), lambda i: (0, 0)),
            pl.BlockSpec((1, 128), lambda i: (0, 0)),
        ],
        out_specs=pl.BlockSpec((P, 128), lambda i: (0, 0)),
        out_shape=jax.ShapeDtypeStruct((P, 128), _f32),
    )(aggs, latents, wg, bg2, wd, bd2, wh, bh2)


def kernel(latents, node_features, senders, receivers, edge_features,
           W_t2g, b_t2g, W_edge, b_edge, W_gnn, b_gnn, W_dec, b_dec,
           W_actor, b_actor, W_critic, b_critic):
    bg2 = b_gnn.reshape(1, DG)
    nf = _node_encode(node_features, W_t2g, b_t2g.reshape(1, DG))
    ee = _edge_proj(edge_features, W_edge, b_edge.reshape(1, DG))

    aggs1, eids, pkds, cnts = _sc_step1(nf, senders, receivers, ee)
    nf2 = _mid(aggs1, nf, W_gnn, bg2)
    aggs2 = _sc_step2(nf2, eids, pkds, cnts, ee, senders, receivers)

    wh = jnp.zeros((DF + DG, 128), _f32)
    wh = wh.at[:, :A].set(W_actor).at[:, A:A + 1].set(W_critic)
    bh = jnp.zeros((1, 128), _f32)
    bh = bh.at[0, :A].set(b_actor).at[0, A:A + 1].set(b_critic)

    heads = _heads(aggs2, latents, W_gnn, bg2, W_dec, b_dec.reshape(1, DG),
                   wh, bh)
    value = heads[:, A:A + 1]
    policy = heads[:, :A]
    return value, policy
